# baseline (speedup 1.0000x reference)
import functools

import jax
import jax.numpy as jnp
from jax.experimental import pallas as pl
from jax.experimental.pallas import tpu as pltpu


def _tcol(ident, row):
    return jax.lax.dot_general(ident, row, (((1,), (1,)), ((), ())),
                               preferred_element_type=jnp.float32)


def _fused_kernel(N, H, W, eps,
                  x_ref, w_ref, b_ref, g_ref, bt_ref,
                  o_ref,
                  xs_ref, y_ref, wb_ref, bc_ref, st_ref, aff_ref):
    p = pl.program_id(0)
    n = pl.program_id(1)
    HW = H * W
    cin = x_ref.shape[1]
    cout = w_ref.shape[0]
    K = w_ref.shape[1]
    pad = xs_ref.shape[-1]

    @pl.when((p == 0) & (n == 0))
    def _prep():
        i = jax.lax.broadcasted_iota(jnp.int32, (K, K), 0)
        j = jax.lax.broadcasted_iota(jnp.int32, (K, K), 1)
        perm = (j == (i % 9) * cin + i // 9).astype(jnp.bfloat16)
        wb_ref[...] = jnp.dot(w_ref[...].astype(jnp.bfloat16), perm,
                              preferred_element_type=jnp.float32
                              ).astype(jnp.bfloat16)
        ident = (jax.lax.broadcasted_iota(jnp.int32, (cout, cout), 0) ==
                 jax.lax.broadcasted_iota(jnp.int32, (cout, cout), 1)
                 ).astype(jnp.float32)
        bc_ref[...] = _tcol(ident, b_ref[...])

    @pl.when(p == 0)
    def _conv_phase():
        xb = x_ref[0].astype(jnp.bfloat16)
        col = jax.lax.broadcasted_iota(jnp.int32, (1, HW), 1) % W
        xl = jnp.where(col == W - 1, jnp.bfloat16(0), xb)
        xr = jnp.where(col == 0, jnp.bfloat16(0), xb)
        buf = n % 2
        for ky in range(3):
            for kx in range(3):
                off = (W + 2) - (ky - 1) * W - kx
                r0 = (ky * 3 + kx) * cin
                src = (xl, xb, xr)[kx]
                if off > 0:
                    xs_ref[buf, r0:r0 + cin, :off] = jnp.zeros(
                        (cin, off), jnp.bfloat16)
                xs_ref[buf, r0:r0 + cin, off:off + HW] = src
                tail = pad - off - HW
                if tail > 0:
                    xs_ref[buf, r0:r0 + cin, off + HW:] = jnp.zeros(
                        (cin, tail), jnp.bfloat16)

        m = jnp.maximum(n - 1, 0)
        acc = jnp.dot(wb_ref[...], xs_ref[(n + 1) % 2, :, W + 1:W + 1 + HW],
                      preferred_element_type=jnp.float32)
        acc = jnp.maximum(acc + bc_ref[...], 0.0)
        y_ref[m] = acc.astype(jnp.bfloat16)

        s = jnp.sum(acc, axis=1, keepdims=True)
        ss = jnp.sum(acc * acc, axis=1, keepdims=True)
        step = jnp.concatenate([s, ss], axis=1)
        @pl.when(n == 1)
        def _():
            st_ref[...] = step
        @pl.when(n > 1)
        def _():
            st_ref[...] = st_ref[...] + step

    @pl.when(p == 1)
    def _apply_phase():
        @pl.when(n == 0)
        def _():
            ident = (jax.lax.broadcasted_iota(jnp.int32, (cout, cout), 0) ==
                     jax.lax.broadcasted_iota(jnp.int32, (cout, cout), 1)
                     ).astype(jnp.float32)
            count = float(N * HW)
            mean = st_ref[:, 0:1] / count
            var = st_ref[:, 1:2] / count - mean * mean
            scale = _tcol(ident, g_ref[...]) * jax.lax.rsqrt(var + eps)
            shift = _tcol(ident, bt_ref[...]) - mean * scale
            aff_ref[...] = jnp.concatenate([scale, shift], axis=1)
        nn = jnp.minimum(n, N - 1)
        o_ref[0] = (y_ref[nn].astype(jnp.float32) * aff_ref[:, 0:1]
                    + aff_ref[:, 1:2])


def kernel(x, conv_w, conv_b, gamma, beta, eps=1e-5):
    N, Cin, H, Wd = x.shape
    Cout = conv_w.shape[0]
    HW = H * Wd
    pad = -(-(HW + 2 * Wd + 2) // 128) * 128

    xf = x.reshape(N, Cin, HW)
    wf = conv_w.reshape(Cout, Cin * 9)
    b1 = conv_b.reshape(1, Cout)
    g1 = gamma.reshape(1, Cout)
    bt1 = beta.reshape(1, Cout)

    out = pl.pallas_call(
        functools.partial(_fused_kernel, N, H, Wd, eps),
        out_shape=jax.ShapeDtypeStruct((N, Cout, HW), jnp.float32),
        grid=(2, N + 1),
        in_specs=[
            pl.BlockSpec((1, Cin, HW),
                         lambda p, n: ((1 - p) * jnp.minimum(n, N - 1)
                                       + p * (N - 1), 0, 0)),
            pl.BlockSpec((Cout, 9 * Cin), lambda p, n: (0, 0)),
            pl.BlockSpec((1, Cout), lambda p, n: (0, 0)),
            pl.BlockSpec((1, Cout), lambda p, n: (0, 0)),
            pl.BlockSpec((1, Cout), lambda p, n: (0, 0)),
        ],
        out_specs=pl.BlockSpec(
            (1, Cout, HW),
            lambda p, n: (p * jnp.minimum(n, N - 1), 0, 0)),
        scratch_shapes=[
            pltpu.VMEM((2, 9 * Cin, pad), jnp.bfloat16),
            pltpu.VMEM((N, Cout, HW), jnp.bfloat16),
            pltpu.VMEM((Cout, 9 * Cin), jnp.bfloat16),
            pltpu.VMEM((Cout, 1), jnp.float32),
            pltpu.VMEM((Cout, 2), jnp.float32),
            pltpu.VMEM((Cout, 2), jnp.float32),
        ],
        compiler_params=pltpu.CompilerParams(
            dimension_semantics=("arbitrary", "arbitrary"),
            vmem_limit_bytes=64 * 1024 * 1024),
    )(xf, wf, b1, g1, bt1)

    return out.reshape(N, Cout, H, Wd)


# speedup vs baseline: 1.0081x; 1.0081x over previous
import functools

import jax
import jax.numpy as jnp
from jax.experimental import pallas as pl
from jax.experimental.pallas import tpu as pltpu


def _tcol(ident, row):
    return jax.lax.dot_general(ident, row, (((1,), (1,)), ((), ())),
                               preferred_element_type=jnp.float32)


def _fused_kernel(N, H, W, eps,
                  x_ref, w_ref, b_ref, g_ref, bt_ref,
                  o_ref,
                  xsa_ref, xsb_ref, y_ref, wb_ref, bc_ref, st_ref,
                  aff_ref):
    p = pl.program_id(0)
    n = pl.program_id(1)
    HW = H * W
    cin = x_ref.shape[1]
    cout = w_ref.shape[0]
    K = w_ref.shape[1]
    pad = xsa_ref.shape[-1]

    @pl.when((p == 0) & (n == 0))
    def _prep():
        i = jax.lax.broadcasted_iota(jnp.int32, (K, K), 0)
        j = jax.lax.broadcasted_iota(jnp.int32, (K, K), 1)
        perm = (j == (i % 9) * cin + i // 9).astype(jnp.bfloat16)
        wb_ref[...] = jnp.dot(w_ref[...].astype(jnp.bfloat16), perm,
                              preferred_element_type=jnp.float32
                              ).astype(jnp.bfloat16)
        ident = (jax.lax.broadcasted_iota(jnp.int32, (cout, cout), 0) ==
                 jax.lax.broadcasted_iota(jnp.int32, (cout, cout), 1)
                 ).astype(jnp.float32)
        bc_ref[...] = _tcol(ident, b_ref[...])

    def _conv_step(build_ref, dot_ref):
        xb = x_ref[0].astype(jnp.bfloat16)
        col = jax.lax.broadcasted_iota(jnp.int32, (1, HW), 1) % W
        xl = jnp.where(col == W - 1, jnp.bfloat16(0), xb)
        xr = jnp.where(col == 0, jnp.bfloat16(0), xb)
        for ky in range(3):
            for kx in range(3):
                off = (W + 2) - (ky - 1) * W - kx
                r0 = (ky * 3 + kx) * cin
                src = (xl, xb, xr)[kx]
                if off > 0:
                    build_ref[r0:r0 + cin, :off] = jnp.zeros(
                        (cin, off), jnp.bfloat16)
                build_ref[r0:r0 + cin, off:off + HW] = src
                tail = pad - off - HW
                if tail > 0:
                    build_ref[r0:r0 + cin, off + HW:] = jnp.zeros(
                        (cin, tail), jnp.bfloat16)

        m = jnp.maximum(n - 1, 0)
        acc = jnp.dot(wb_ref[...], dot_ref[:, W + 1:W + 1 + HW],
                      preferred_element_type=jnp.float32)
        acc = jnp.maximum(acc + bc_ref[...], 0.0)
        y_ref[m] = acc.astype(jnp.bfloat16)

        s = jnp.sum(acc, axis=1, keepdims=True)
        ss = jnp.sum(acc * acc, axis=1, keepdims=True)
        step = jnp.concatenate([s, ss], axis=1)
        @pl.when(n == 1)
        def _():
            st_ref[...] = step
        @pl.when(n > 1)
        def _():
            st_ref[...] = st_ref[...] + step

    @pl.when(p == 0)
    def _conv_phase():
        @pl.when(n % 2 == 0)
        def _():
            _conv_step(xsa_ref, xsb_ref)
        @pl.when(n % 2 == 1)
        def _():
            _conv_step(xsb_ref, xsa_ref)

    @pl.when(p == 1)
    def _apply_phase():
        @pl.when(n == 0)
        def _():
            ident = (jax.lax.broadcasted_iota(jnp.int32, (cout, cout), 0) ==
                     jax.lax.broadcasted_iota(jnp.int32, (cout, cout), 1)
                     ).astype(jnp.float32)
            count = float(N * HW)
            mean = st_ref[:, 0:1] / count
            var = st_ref[:, 1:2] / count - mean * mean
            scale = _tcol(ident, g_ref[...]) * jax.lax.rsqrt(var + eps)
            shift = _tcol(ident, bt_ref[...]) - mean * scale
            aff_ref[...] = jnp.concatenate([scale, shift], axis=1)
        nn = jnp.minimum(n, N - 1)
        o_ref[0] = (y_ref[nn].astype(jnp.float32) * aff_ref[:, 0:1]
                    + aff_ref[:, 1:2])


def kernel(x, conv_w, conv_b, gamma, beta, eps=1e-5):
    N, Cin, H, Wd = x.shape
    Cout = conv_w.shape[0]
    HW = H * Wd
    pad = -(-(HW + 2 * Wd + 2) // 128) * 128

    xf = x.reshape(N, Cin, HW)
    wf = conv_w.reshape(Cout, Cin * 9)
    b1 = conv_b.reshape(1, Cout)
    g1 = gamma.reshape(1, Cout)
    bt1 = beta.reshape(1, Cout)

    out = pl.pallas_call(
        functools.partial(_fused_kernel, N, H, Wd, eps),
        out_shape=jax.ShapeDtypeStruct((N, Cout, HW), jnp.float32),
        grid=(2, N + 1),
        in_specs=[
            pl.BlockSpec((1, Cin, HW),
                         lambda p, n: ((1 - p) * jnp.minimum(n, N - 1)
                                       + p * (N - 1), 0, 0)),
            pl.BlockSpec((Cout, 9 * Cin), lambda p, n: (0, 0)),
            pl.BlockSpec((1, Cout), lambda p, n: (0, 0)),
            pl.BlockSpec((1, Cout), lambda p, n: (0, 0)),
            pl.BlockSpec((1, Cout), lambda p, n: (0, 0)),
        ],
        out_specs=pl.BlockSpec(
            (1, Cout, HW),
            lambda p, n: (p * jnp.minimum(n, N - 1), 0, 0)),
        scratch_shapes=[
            pltpu.VMEM((9 * Cin, pad), jnp.bfloat16),
            pltpu.VMEM((9 * Cin, pad), jnp.bfloat16),
            pltpu.VMEM((N, Cout, HW), jnp.bfloat16),
            pltpu.VMEM((Cout, 9 * Cin), jnp.bfloat16),
            pltpu.VMEM((Cout, 1), jnp.float32),
            pltpu.VMEM((Cout, 2), jnp.float32),
            pltpu.VMEM((Cout, 2), jnp.float32),
        ],
        compiler_params=pltpu.CompilerParams(
            dimension_semantics=("arbitrary", "arbitrary"),
            vmem_limit_bytes=64 * 1024 * 1024),
    )(xf, wf, b1, g1, bt1)

    return out.reshape(N, Cout, H, Wd)


# speedup vs baseline: 1.1692x; 1.1599x over previous
import functools

import jax
import jax.numpy as jnp
from jax.experimental import pallas as pl
from jax.experimental.pallas import tpu as pltpu


def _tcol(ident, row):
    return jax.lax.dot_general(ident, row, (((1,), (1,)), ((), ())),
                               preferred_element_type=jnp.float32)


def _fused_kernel(N, H, W, eps,
                  x_ref, w_ref, b_ref, g_ref, bt_ref,
                  o_ref,
                  xs_ref, y_ref, wb_ref, bc_ref, st_ref, aff_ref):
    p = pl.program_id(0)
    n = pl.program_id(1)
    HW = H * W
    cin = x_ref.shape[1]
    cout = w_ref.shape[0]
    K = w_ref.shape[1]
    pad = xs_ref.shape[-1]

    @pl.when((p == 0) & (n == 0))
    def _prep():
        i = jax.lax.broadcasted_iota(jnp.int32, (K, K), 0)
        j = jax.lax.broadcasted_iota(jnp.int32, (K, K), 1)
        t = i % 9
        perm = (j == ((t % 3) * 3 + t // 3) * cin + i // 9
                ).astype(jnp.bfloat16)
        wb_ref[...] = jnp.dot(w_ref[...].astype(jnp.bfloat16), perm,
                              preferred_element_type=jnp.float32
                              ).astype(jnp.bfloat16)
        ident = (jax.lax.broadcasted_iota(jnp.int32, (cout, cout), 0) ==
                 jax.lax.broadcasted_iota(jnp.int32, (cout, cout), 1)
                 ).astype(jnp.float32)
        bc_ref[...] = _tcol(ident, b_ref[...])

    @pl.when(p == 0)
    def _conv_phase():
        xb = x_ref[0].astype(jnp.bfloat16)
        for ky in range(3):
            off = (2 - ky) * W
            r0 = ky * cin
            if off > 0:
                xs_ref[r0:r0 + cin, :off] = jnp.zeros((cin, off),
                                                      jnp.bfloat16)
            xs_ref[r0:r0 + cin, off:off + HW] = xb
            tail = pad - off - HW
            if tail > 0:
                xs_ref[r0:r0 + cin, off + HW:] = jnp.zeros((cin, tail),
                                                           jnp.bfloat16)

        col = jax.lax.broadcasted_iota(jnp.int32, (1, HW), 1) % W
        acc = jnp.zeros((cout, HW), jnp.float32)
        for kx in range(3):
            sl = xs_ref[:, W - 1 + kx:W - 1 + kx + HW]
            if kx == 0:
                sl = jnp.where(col == 0, jnp.bfloat16(0), sl)
            elif kx == 2:
                sl = jnp.where(col == W - 1, jnp.bfloat16(0), sl)
            acc = acc + jnp.dot(wb_ref[:, kx * 3 * cin:(kx + 1) * 3 * cin],
                                sl, preferred_element_type=jnp.float32)

        acc = jnp.maximum(acc + bc_ref[...], 0.0)
        y_ref[n] = acc.astype(jnp.bfloat16)

        s = jnp.sum(acc, axis=1, keepdims=True)
        ss = jnp.sum(acc * acc, axis=1, keepdims=True)
        step = jnp.concatenate([s, ss], axis=1)
        @pl.when(n == 0)
        def _():
            st_ref[...] = step
        @pl.when(n > 0)
        def _():
            st_ref[...] = st_ref[...] + step

    @pl.when(p == 1)
    def _apply_phase():
        @pl.when(n == 0)
        def _():
            ident = (jax.lax.broadcasted_iota(jnp.int32, (cout, cout), 0) ==
                     jax.lax.broadcasted_iota(jnp.int32, (cout, cout), 1)
                     ).astype(jnp.float32)
            count = float(N * HW)
            mean = st_ref[:, 0:1] / count
            var = st_ref[:, 1:2] / count - mean * mean
            scale = _tcol(ident, g_ref[...]) * jax.lax.rsqrt(var + eps)
            shift = _tcol(ident, bt_ref[...]) - mean * scale
            aff_ref[...] = jnp.concatenate([scale, shift], axis=1)
        o_ref[0] = (y_ref[n].astype(jnp.float32) * aff_ref[:, 0:1]
                    + aff_ref[:, 1:2])


def kernel(x, conv_w, conv_b, gamma, beta, eps=1e-5):
    N, Cin, H, Wd = x.shape
    Cout = conv_w.shape[0]
    HW = H * Wd
    pad = -(-(HW + 2 * Wd) // 128) * 128

    xf = x.reshape(N, Cin, HW)
    wf = conv_w.reshape(Cout, Cin * 9)
    b1 = conv_b.reshape(1, Cout)
    g1 = gamma.reshape(1, Cout)
    bt1 = beta.reshape(1, Cout)

    out = pl.pallas_call(
        functools.partial(_fused_kernel, N, H, Wd, eps),
        out_shape=jax.ShapeDtypeStruct((N, Cout, HW), jnp.float32),
        grid=(2, N),
        in_specs=[
            pl.BlockSpec((1, Cin, HW),
                         lambda p, n: ((1 - p) * n + p * (N - 1), 0, 0)),
            pl.BlockSpec((Cout, 9 * Cin), lambda p, n: (0, 0)),
            pl.BlockSpec((1, Cout), lambda p, n: (0, 0)),
            pl.BlockSpec((1, Cout), lambda p, n: (0, 0)),
            pl.BlockSpec((1, Cout), lambda p, n: (0, 0)),
        ],
        out_specs=pl.BlockSpec((1, Cout, HW), lambda p, n: (p * n, 0, 0)),
        scratch_shapes=[
            pltpu.VMEM((3 * Cin, pad), jnp.bfloat16),
            pltpu.VMEM((N, Cout, HW), jnp.bfloat16),
            pltpu.VMEM((Cout, 9 * Cin), jnp.bfloat16),
            pltpu.VMEM((Cout, 1), jnp.float32),
            pltpu.VMEM((Cout, 2), jnp.float32),
            pltpu.VMEM((Cout, 2), jnp.float32),
        ],
        compiler_params=pltpu.CompilerParams(
            dimension_semantics=("arbitrary", "arbitrary"),
            vmem_limit_bytes=64 * 1024 * 1024),
    )(xf, wf, b1, g1, bt1)

    return out.reshape(N, Cout, H, Wd)


# speedup vs baseline: 1.2015x; 1.0276x over previous
import functools

import jax
import jax.numpy as jnp
from jax.experimental import pallas as pl
from jax.experimental.pallas import tpu as pltpu


def _tcol(ident, row):
    return jax.lax.dot_general(ident, row, (((1,), (1,)), ((), ())),
                               preferred_element_type=jnp.float32)


def _fused_kernel(N, H, W, eps,
                  x_ref, w_ref, b_ref, g_ref, bt_ref,
                  o_ref,
                  xs_ref, y_ref, wb_ref, bc_ref, st_ref, aff_ref):
    p = pl.program_id(0)
    n = pl.program_id(1)
    HW = H * W
    cin = x_ref.shape[1]
    cout = w_ref.shape[0]
    K = w_ref.shape[1]
    pad = xs_ref.shape[-1]

    @pl.when((p == 0) & (n == 0))
    def _prep():
        i = jax.lax.broadcasted_iota(jnp.int32, (K, K), 0)
        j = jax.lax.broadcasted_iota(jnp.int32, (K, K), 1)
        t = i % 9
        perm = (j == ((t % 3) * 3 + t // 3) * cin + i // 9
                ).astype(jnp.bfloat16)
        wb_ref[...] = jnp.dot(w_ref[...].astype(jnp.bfloat16), perm,
                              preferred_element_type=jnp.float32
                              ).astype(jnp.bfloat16)
        ident = (jax.lax.broadcasted_iota(jnp.int32, (cout, cout), 0) ==
                 jax.lax.broadcasted_iota(jnp.int32, (cout, cout), 1)
                 ).astype(jnp.float32)
        bc_ref[...] = _tcol(ident, b_ref[...])

    @pl.when(p == 0)
    def _conv_phase():
        col = jax.lax.broadcasted_iota(jnp.int32, (1, HW), 1) % W
        for s in range(2):
            xb = x_ref[s].astype(jnp.bfloat16)
            for ky in range(3):
                off = (2 - ky) * W
                r0 = ky * cin
                if off > 0:
                    xs_ref[r0:r0 + cin, :off] = jnp.zeros((cin, off),
                                                          jnp.bfloat16)
                xs_ref[r0:r0 + cin, off:off + HW] = xb
                tail = pad - off - HW
                if tail > 0:
                    xs_ref[r0:r0 + cin, off + HW:] = jnp.zeros(
                        (cin, tail), jnp.bfloat16)

            acc = jnp.zeros((cout, HW), jnp.float32)
            for kx in range(3):
                sl = xs_ref[:, W - 1 + kx:W - 1 + kx + HW]
                if kx == 0:
                    sl = jnp.where(col == 0, jnp.bfloat16(0), sl)
                elif kx == 2:
                    sl = jnp.where(col == W - 1, jnp.bfloat16(0), sl)
                acc = acc + jnp.dot(
                    wb_ref[:, kx * 3 * cin:(kx + 1) * 3 * cin],
                    sl, preferred_element_type=jnp.float32)

            acc = jnp.maximum(acc + bc_ref[...], 0.0)
            y_ref[2 * n + s] = acc.astype(jnp.bfloat16)

            s1 = jnp.sum(acc, axis=1, keepdims=True)
            ss = jnp.sum(acc * acc, axis=1, keepdims=True)
            step = jnp.concatenate([s1, ss], axis=1)
            if s == 0:
                @pl.when(n == 0)
                def _():
                    st_ref[...] = step
                @pl.when(n > 0)
                def _():
                    st_ref[...] = st_ref[...] + step
            else:
                st_ref[...] = st_ref[...] + step

    @pl.when(p == 1)
    def _apply_phase():
        @pl.when(n == 0)
        def _():
            ident = (jax.lax.broadcasted_iota(jnp.int32, (cout, cout), 0) ==
                     jax.lax.broadcasted_iota(jnp.int32, (cout, cout), 1)
                     ).astype(jnp.float32)
            count = float(N * HW)
            mean = st_ref[:, 0:1] / count
            var = st_ref[:, 1:2] / count - mean * mean
            scale = _tcol(ident, g_ref[...]) * jax.lax.rsqrt(var + eps)
            shift = _tcol(ident, bt_ref[...]) - mean * scale
            aff_ref[...] = jnp.concatenate([scale, shift], axis=1)
        for s in range(2):
            o_ref[s] = (y_ref[2 * n + s].astype(jnp.float32)
                        * aff_ref[:, 0:1] + aff_ref[:, 1:2])


def kernel(x, conv_w, conv_b, gamma, beta, eps=1e-5):
    N, Cin, H, Wd = x.shape
    Cout = conv_w.shape[0]
    HW = H * Wd
    NB = N // 2
    pad = -(-(HW + 2 * Wd) // 128) * 128

    xf = x.reshape(N, Cin, HW)
    wf = conv_w.reshape(Cout, Cin * 9)
    b1 = conv_b.reshape(1, Cout)
    g1 = gamma.reshape(1, Cout)
    bt1 = beta.reshape(1, Cout)

    out = pl.pallas_call(
        functools.partial(_fused_kernel, N, H, Wd, eps),
        out_shape=jax.ShapeDtypeStruct((N, Cout, HW), jnp.float32),
        grid=(2, NB),
        in_specs=[
            pl.BlockSpec((2, Cin, HW),
                         lambda p, n: ((1 - p) * n + p * (NB - 1), 0, 0)),
            pl.BlockSpec((Cout, 9 * Cin), lambda p, n: (0, 0)),
            pl.BlockSpec((1, Cout), lambda p, n: (0, 0)),
            pl.BlockSpec((1, Cout), lambda p, n: (0, 0)),
            pl.BlockSpec((1, Cout), lambda p, n: (0, 0)),
        ],
        out_specs=pl.BlockSpec((2, Cout, HW), lambda p, n: (p * n, 0, 0)),
        scratch_shapes=[
            pltpu.VMEM((3 * Cin, pad), jnp.bfloat16),
            pltpu.VMEM((N, Cout, HW), jnp.bfloat16),
            pltpu.VMEM((Cout, 9 * Cin), jnp.bfloat16),
            pltpu.VMEM((Cout, 1), jnp.float32),
            pltpu.VMEM((Cout, 2), jnp.float32),
            pltpu.VMEM((Cout, 2), jnp.float32),
        ],
        compiler_params=pltpu.CompilerParams(
            dimension_semantics=("arbitrary", "arbitrary"),
            vmem_limit_bytes=64 * 1024 * 1024),
    )(xf, wf, b1, g1, bt1)

    return out.reshape(N, Cout, H, Wd)
